# baseline (speedup 1.0000x reference)
import functools

import jax
import jax.numpy as jnp
from jax import lax
from jax.experimental import pallas as pl
from jax.experimental.pallas import tpu as pltpu
from jax.experimental.pallas import tpu_sc as plsc

N = 10000
E = 320000
D = 128
H1 = 512
H2 = 128
OUT = 40

NC = 2
NS = 16
NW = NC * NS
EPW = E // NW
K = 80
NB = EPW // K
ACC_N = 10240
DEG_W = 16
CH = 25
NCH = NB // CH
KD = 125

_mesh = plsc.VectorSubcoreMesh(core_axis_name="c", subcore_axis_name="s")


def _zero_rows(ref, nrows, width):
    z16 = jnp.zeros((16,), jnp.float32)
    cpr = width // 16

    def body(i, _):
        r = i // cpr
        c = (i % cpr) * 16
        ref[r, pl.ds(c, 16)] = z16
        return 0

    lax.fori_loop(0, nrows * cpr, body, 0)


def _wid():
    return lax.axis_index("s") * NC + lax.axis_index("c")


@functools.partial(
    pl.kernel,
    mesh=_mesh,
    out_type=jax.ShapeDtypeStruct((NC, ACC_N), jnp.float32),
    scratch_types=[
        pltpu.VMEM((EPW // KD, KD), jnp.int32),
        pltpu.VMEM((KD,), jnp.float32),
        pltpu.VMEM((ACC_N // NS,), jnp.float32),
        pltpu.VMEM_SHARED((ACC_N,), jnp.float32),
    ],
)
def _sc_degree(dst_hbm, out_hbm, dst_v, ones_v, zb_v, acc):
    c = lax.axis_index("c")
    s = lax.axis_index("s")
    wid = _wid()

    pltpu.sync_copy(dst_hbm.at[wid], dst_v)

    o16 = jnp.ones((16,), jnp.float32)
    z16 = jnp.zeros((16,), jnp.float32)
    for i in range(KD // 16):
        ones_v[pl.ds(i * 16, 16)] = o16
    if KD % 16:
        ones_v[pl.ds(KD - 16, 16)] = o16

    def zb(i, _):
        zb_v[pl.ds(i * 16, 16)] = z16
        return 0

    lax.fori_loop(0, (ACC_N // NS) // 16, zb, 0)
    base = s * (ACC_N // NS)
    pltpu.sync_copy(zb_v, acc.at[pl.ds(base, ACC_N // NS)])
    plsc.subcore_barrier()

    def body(j, _):
        pltpu.sync_copy(ones_v, acc.at[dst_v.at[j]], add=True)
        return 0

    lax.fori_loop(0, EPW // KD, body, 0)

    plsc.subcore_barrier()
    pltpu.sync_copy(
        acc.at[pl.ds(base, ACC_N // NS)],
        out_hbm.at[c, pl.ds(base, ACC_N // NS)],
    )


@functools.partial(
    pl.kernel,
    mesh=_mesh,
    out_type=jax.ShapeDtypeStruct((NC, ACC_N, D), jnp.float32),
    scratch_types=[
        pltpu.VMEM((2, CH, K), jnp.int32),
        pltpu.VMEM((2, CH, K), jnp.int32),
        pltpu.VMEM((K, D), jnp.float32),
        pltpu.VMEM((K, D), jnp.float32),
        pltpu.VMEM((K, D), jnp.float32),
        pltpu.VMEM_SHARED((ACC_N, D), jnp.float32),
        pltpu.SemaphoreType.DMA,
        pltpu.SemaphoreType.DMA,
        pltpu.SemaphoreType.DMA,
        pltpu.SemaphoreType.DMA,
        pltpu.SemaphoreType.DMA,
        pltpu.SemaphoreType.DMA,
        pltpu.SemaphoreType.DMA,
        pltpu.SemaphoreType.DMA,
    ],
)
def _sc_scatter(u_hbm, src_hbm, dst_hbm, out_hbm, src_v, dst_v, b0, b1, b2,
                acc, g0, g1, g2, s0, s1, s2, i0, i1):
    c = lax.axis_index("c")
    s = lax.axis_index("s")
    wid = _wid()
    bufs = [b0, b1, b2]
    gs = [g0, g1, g2]
    ss = [s0, s1, s2]
    isems = [i0, i1]

    _zero_rows(b0, K, D)
    base = s * (ACC_N // NS)
    for i in range(8):
        pltpu.sync_copy(b0, acc.at[pl.ds(base + i * K, K)])
    plsc.subcore_barrier()

    pltpu.async_copy(src_hbm.at[wid, 0], src_v.at[0], isems[0])
    pltpu.async_copy(dst_hbm.at[wid, 0], dst_v.at[0], isems[0])

    def chunk_body(ch, p):
        src = src_v.at[p]
        dst = dst_v.at[p]
        pltpu.make_async_copy(src_hbm.at[wid, ch], src, isems[p]).wait()
        pltpu.make_async_copy(dst_hbm.at[wid, ch], dst, isems[p]).wait()

        @pl.when(ch + 1 < NCH)
        def _():
            pltpu.async_copy(src_hbm.at[wid, ch + 1], src_v.at[1 - p], isems[1 - p])
            pltpu.async_copy(dst_hbm.at[wid, ch + 1], dst_v.at[1 - p], isems[1 - p])

        pltpu.async_copy(u_hbm.at[src.at[0]], bufs[0], gs[0])
        pltpu.async_copy(u_hbm.at[src.at[1]], bufs[1], gs[1])
        for j in range(CH):
            b = j % 3
            if j >= 1:
                pltpu.make_async_copy(
                    bufs[(j - 1) % 3], acc.at[dst.at[j - 1]], ss[(j - 1) % 3]
                ).wait()
            if j + 2 < CH:
                bb = (j + 2) % 3
                pltpu.async_copy(u_hbm.at[src.at[j + 2]], bufs[bb], gs[bb])
            pltpu.make_async_copy(u_hbm.at[src.at[j]], bufs[b], gs[b]).wait()
            pltpu.async_copy(bufs[b], acc.at[dst.at[j]], ss[b], add=True)
        pltpu.make_async_copy(
            bufs[(CH - 1) % 3], acc.at[dst.at[CH - 1]], ss[(CH - 1) % 3]
        ).wait()

    def pair(m, _):
        chunk_body(2 * m, 0)
        chunk_body(2 * m + 1, 1)
        return 0

    lax.fori_loop(0, NCH // 2, pair, 0)
    if NCH % 2:
        chunk_body(NCH - 1, 0)

    plsc.subcore_barrier()
    pltpu.sync_copy(
        acc.at[pl.ds(base, ACC_N // NS)],
        out_hbm.at[c, pl.ds(base, ACC_N // NS)],
    )


_BN = 2000
_NBLK = N // _BN


def _tc1_body(degp_ref, x_ref, dinv_ref, u1_ref):
    deg = degp_ref[:, 0] + degp_ref[:, 1] + 1.0
    dinv = lax.rsqrt(deg)
    dinv_ref[...] = jnp.broadcast_to(dinv[:, None], (_BN, 8))
    u1_ref[...] = x_ref[...] * dinv[:, None]


def _tc1(degp, x):
    return pl.pallas_call(
        _tc1_body,
        grid=(_NBLK,),
        in_specs=[
            pl.BlockSpec((_BN, NC), lambda i: (i, 0)),
            pl.BlockSpec((_BN, D), lambda i: (i, 0)),
        ],
        out_specs=[
            pl.BlockSpec((_BN, 8), lambda i: (i, 0)),
            pl.BlockSpec((_BN, D), lambda i: (i, 0)),
        ],
        out_shape=[
            jax.ShapeDtypeStruct((N, 8), jnp.float32),
            jax.ShapeDtypeStruct((N, D), jnp.float32),
        ],
    )(degp, x)


def _tc2_body(p_ref, x_ref, dinv_ref, W1_ref, b1_ref, W2_ref, a_ref, t2_ref, u2_ref):
    dinv = dinv_ref[:, 0:1]
    ssum = p_ref[0] + p_ref[1]
    a1 = dinv * ssum + (dinv * dinv) * x_ref[...]
    f1 = jnp.dot(a1, W1_ref[...], preferred_element_type=jnp.float32) + b1_ref[...]
    al = a_ref[0, 0]
    f1 = jnp.where(f1 >= 0, f1, al * f1)
    t2 = jnp.dot(f1, W2_ref[...], preferred_element_type=jnp.float32)
    t2_ref[...] = t2
    u2_ref[...] = t2 * dinv


def _tc2(p, x, dinv, W1, b1, W2, a):
    return pl.pallas_call(
        _tc2_body,
        grid=(_NBLK,),
        in_specs=[
            pl.BlockSpec((NC, _BN, D), lambda i: (0, i, 0)),
            pl.BlockSpec((_BN, D), lambda i: (i, 0)),
            pl.BlockSpec((_BN, 8), lambda i: (i, 0)),
            pl.BlockSpec((D, H1), lambda i: (0, 0)),
            pl.BlockSpec((1, H1), lambda i: (0, 0)),
            pl.BlockSpec((H1, H2), lambda i: (0, 0)),
            pl.BlockSpec((1, 1), lambda i: (0, 0)),
        ],
        out_specs=[
            pl.BlockSpec((_BN, H2), lambda i: (i, 0)),
            pl.BlockSpec((_BN, H2), lambda i: (i, 0)),
        ],
        out_shape=[
            jax.ShapeDtypeStruct((N, H2), jnp.float32),
            jax.ShapeDtypeStruct((N, H2), jnp.float32),
        ],
    )(p, x, dinv, W1, b1, W2, a)


def _tc3_body(p_ref, t2_ref, dinv_ref, b2_ref, fw_ref, fb_ref, a_ref, out_ref):
    dinv = dinv_ref[:, 0:1]
    ssum = p_ref[0] + p_ref[1]
    a2 = dinv * ssum + (dinv * dinv) * t2_ref[...] + b2_ref[...]
    al = a_ref[0, 0]
    f2 = jnp.where(a2 >= 0, a2, al * a2)
    out_ref[...] = jnp.dot(f2, fw_ref[...], preferred_element_type=jnp.float32) + fb_ref[...]


def _tc3(p, t2, dinv, b2, fw, fb, a):
    return pl.pallas_call(
        _tc3_body,
        grid=(_NBLK,),
        in_specs=[
            pl.BlockSpec((NC, _BN, D), lambda i: (0, i, 0)),
            pl.BlockSpec((_BN, H2), lambda i: (i, 0)),
            pl.BlockSpec((_BN, 8), lambda i: (i, 0)),
            pl.BlockSpec((1, H2), lambda i: (0, 0)),
            pl.BlockSpec((H2, 128), lambda i: (0, 0)),
            pl.BlockSpec((1, 128), lambda i: (0, 0)),
            pl.BlockSpec((1, 1), lambda i: (0, 0)),
        ],
        out_specs=pl.BlockSpec((_BN, 128), lambda i: (i, 0)),
        out_shape=jax.ShapeDtypeStruct((N, 128), jnp.float32),
    )(p, t2, dinv, b2, fw, fb, a)


def kernel(x, edge_index, W1, b1, W2, b2, prelu_a, fc_W, fc_b):
    dst_r = edge_index[1].reshape(NW, EPW // KD, KD)
    src_c = edge_index[0].reshape(NW, NCH, CH, K)
    dst_c = edge_index[1].reshape(NW, NCH, CH, K)
    a = jnp.reshape(prelu_a, (1, 1)).astype(jnp.float32)
    b1r = b1.reshape(1, H1)
    b2r = b2.reshape(1, H2)
    fw_pad = jnp.zeros((H2, 128), jnp.float32).at[:, :OUT].set(fc_W)
    fb_pad = jnp.zeros((1, 128), jnp.float32).at[:, :OUT].set(fc_b)

    degp = _sc_degree(dst_r)
    dinv, u1 = _tc1(jnp.transpose(degp), x)
    p1 = _sc_scatter(u1, src_c, dst_c)
    t2, u2 = _tc2(p1, x, dinv, W1, b1r, W2, a)
    p2 = _sc_scatter(u2, src_c, dst_c)
    logits_pad = _tc3(p2, t2, dinv, b2r, fw_pad, fb_pad, a)
    return logits_pad[:, :OUT]


# speedup vs baseline: 1.0025x; 1.0025x over previous
import functools

import jax
import jax.numpy as jnp
from jax import lax
from jax.experimental import pallas as pl
from jax.experimental.pallas import tpu as pltpu
from jax.experimental.pallas import tpu_sc as plsc

N = 10000
E = 320000
D = 128
H1 = 512
H2 = 128
OUT = 40

NC = 2
NS = 16
NW = NC * NS
EPW = E // NW
K = 100
NB = EPW // K
ACC_N = 10240
CH = 25
NCH = NB // CH
KD = 125

_mesh = plsc.VectorSubcoreMesh(core_axis_name="c", subcore_axis_name="s")


def _zero_rows(ref, nrows, width):
    z16 = jnp.zeros((16,), jnp.float32)
    cpr = width // 16

    def body(i, _):
        r = i // cpr
        c = (i % cpr) * 16
        ref[r, pl.ds(c, 16)] = z16
        return 0

    lax.fori_loop(0, nrows * cpr, body, 0)


def _wid():
    return lax.axis_index("s") * NC + lax.axis_index("c")


@functools.partial(
    pl.kernel,
    mesh=_mesh,
    out_type=jax.ShapeDtypeStruct((NC, ACC_N), jnp.float32),
    scratch_types=[
        pltpu.VMEM((EPW // KD, KD), jnp.int32),
        pltpu.VMEM((KD,), jnp.float32),
        pltpu.VMEM((ACC_N // NS,), jnp.float32),
        pltpu.VMEM_SHARED((ACC_N,), jnp.float32),
    ],
)
def _sc_degree(dst_hbm, out_hbm, dst_v, ones_v, zb_v, acc):
    c = lax.axis_index("c")
    s = lax.axis_index("s")
    wid = _wid()

    pltpu.sync_copy(dst_hbm.at[wid], dst_v)

    o16 = jnp.ones((16,), jnp.float32)
    z16 = jnp.zeros((16,), jnp.float32)
    for i in range(KD // 16):
        ones_v[pl.ds(i * 16, 16)] = o16
    if KD % 16:
        ones_v[pl.ds(KD - 16, 16)] = o16

    def zb(i, _):
        zb_v[pl.ds(i * 16, 16)] = z16
        return 0

    lax.fori_loop(0, (ACC_N // NS) // 16, zb, 0)
    base = s * (ACC_N // NS)
    pltpu.sync_copy(zb_v, acc.at[pl.ds(base, ACC_N // NS)])
    plsc.subcore_barrier()

    def body(j, _):
        pltpu.sync_copy(ones_v, acc.at[dst_v.at[j]], add=True)
        return 0

    lax.fori_loop(0, EPW // KD, body, 0)

    plsc.subcore_barrier()
    pltpu.sync_copy(
        acc.at[pl.ds(base, ACC_N // NS)],
        out_hbm.at[c, pl.ds(base, ACC_N // NS)],
    )


@functools.partial(
    pl.kernel,
    mesh=_mesh,
    out_type=jax.ShapeDtypeStruct((NC, ACC_N, D), jnp.float32),
    scratch_types=[
        pltpu.VMEM((CH, K), jnp.int32),
        pltpu.VMEM((CH, K), jnp.int32),
        pltpu.VMEM((K, D), jnp.float32),
        pltpu.VMEM((K, D), jnp.float32),
        pltpu.VMEM((K, D), jnp.float32),
        pltpu.VMEM_SHARED((ACC_N, D), jnp.float32),
        pltpu.SemaphoreType.DMA,
        pltpu.SemaphoreType.DMA,
        pltpu.SemaphoreType.DMA,
        pltpu.SemaphoreType.DMA,
        pltpu.SemaphoreType.DMA,
        pltpu.SemaphoreType.DMA,
    ],
)
def _sc_scatter(u_hbm, src_hbm, dst_hbm, out_hbm, src_v, dst_v, b0, b1, b2,
                acc, g0, g1, g2, s0, s1, s2):
    c = lax.axis_index("c")
    s = lax.axis_index("s")
    wid = _wid()
    bufs = [b0, b1, b2]
    gs = [g0, g1, g2]
    ss = [s0, s1, s2]

    _zero_rows(b0, K, D)
    base = s * (ACC_N // NS)
    for i in range(6):
        pltpu.sync_copy(b0, acc.at[pl.ds(base + i * K, K)])
    pltpu.sync_copy(b0.at[pl.ds(0, 40)], acc.at[pl.ds(base + 6 * K, 40)])
    plsc.subcore_barrier()

    def chunk(ch, _):
        pltpu.async_copy(src_hbm.at[wid, ch], src_v, g0)
        pltpu.async_copy(dst_hbm.at[wid, ch], dst_v, g1)
        pltpu.make_async_copy(src_hbm.at[wid, ch], src_v, g0).wait()
        pltpu.make_async_copy(dst_hbm.at[wid, ch], dst_v, g1).wait()
        src = src_v
        dst = dst_v
        pltpu.async_copy(u_hbm.at[src.at[0]], bufs[0], gs[0])
        pltpu.async_copy(u_hbm.at[src.at[1]], bufs[1], gs[1])
        for j in range(CH):
            b = j % 3
            if j >= 1:
                pltpu.make_async_copy(
                    bufs[(j - 1) % 3], acc.at[dst.at[j - 1]], ss[(j - 1) % 3]
                ).wait()
            if j + 2 < CH:
                bb = (j + 2) % 3
                pltpu.async_copy(u_hbm.at[src.at[j + 2]], bufs[bb], gs[bb])
            pltpu.make_async_copy(u_hbm.at[src.at[j]], bufs[b], gs[b]).wait()
            pltpu.async_copy(bufs[b], acc.at[dst.at[j]], ss[b], add=True)
        pltpu.make_async_copy(
            bufs[(CH - 1) % 3], acc.at[dst.at[CH - 1]], ss[(CH - 1) % 3]
        ).wait()
        return 0

    lax.fori_loop(0, NCH, chunk, 0)

    plsc.subcore_barrier()
    pltpu.sync_copy(
        acc.at[pl.ds(base, ACC_N // NS)],
        out_hbm.at[c, pl.ds(base, ACC_N // NS)],
    )


_BN = 2000
_NBLK = N // _BN


def _tc1_body(degp_ref, x_ref, dinv_ref, u1_ref):
    deg = degp_ref[:, 0] + degp_ref[:, 1] + 1.0
    dinv = lax.rsqrt(deg)
    dinv_ref[...] = jnp.broadcast_to(dinv[:, None], (_BN, 8))
    u1_ref[...] = x_ref[...] * dinv[:, None]


def _tc1(degp, x):
    return pl.pallas_call(
        _tc1_body,
        grid=(_NBLK,),
        in_specs=[
            pl.BlockSpec((_BN, NC), lambda i: (i, 0)),
            pl.BlockSpec((_BN, D), lambda i: (i, 0)),
        ],
        out_specs=[
            pl.BlockSpec((_BN, 8), lambda i: (i, 0)),
            pl.BlockSpec((_BN, D), lambda i: (i, 0)),
        ],
        out_shape=[
            jax.ShapeDtypeStruct((N, 8), jnp.float32),
            jax.ShapeDtypeStruct((N, D), jnp.float32),
        ],
    )(degp, x)


def _tc2_body(p_ref, x_ref, dinv_ref, W1_ref, b1_ref, W2_ref, a_ref, t2_ref, u2_ref):
    dinv = dinv_ref[:, 0:1]
    ssum = p_ref[0] + p_ref[1]
    a1 = dinv * ssum + (dinv * dinv) * x_ref[...]
    f1 = jnp.dot(a1, W1_ref[...], preferred_element_type=jnp.float32) + b1_ref[...]
    al = a_ref[0, 0]
    f1 = jnp.where(f1 >= 0, f1, al * f1)
    t2 = jnp.dot(f1, W2_ref[...], preferred_element_type=jnp.float32)
    t2_ref[...] = t2
    u2_ref[...] = t2 * dinv


def _tc2(p, x, dinv, W1, b1, W2, a):
    return pl.pallas_call(
        _tc2_body,
        grid=(_NBLK,),
        in_specs=[
            pl.BlockSpec((NC, _BN, D), lambda i: (0, i, 0)),
            pl.BlockSpec((_BN, D), lambda i: (i, 0)),
            pl.BlockSpec((_BN, 8), lambda i: (i, 0)),
            pl.BlockSpec((D, H1), lambda i: (0, 0)),
            pl.BlockSpec((1, H1), lambda i: (0, 0)),
            pl.BlockSpec((H1, H2), lambda i: (0, 0)),
            pl.BlockSpec((1, 1), lambda i: (0, 0)),
        ],
        out_specs=[
            pl.BlockSpec((_BN, H2), lambda i: (i, 0)),
            pl.BlockSpec((_BN, H2), lambda i: (i, 0)),
        ],
        out_shape=[
            jax.ShapeDtypeStruct((N, H2), jnp.float32),
            jax.ShapeDtypeStruct((N, H2), jnp.float32),
        ],
    )(p, x, dinv, W1, b1, W2, a)


def _tc3_body(p_ref, t2_ref, dinv_ref, b2_ref, fw_ref, fb_ref, a_ref, out_ref):
    dinv = dinv_ref[:, 0:1]
    ssum = p_ref[0] + p_ref[1]
    a2 = dinv * ssum + (dinv * dinv) * t2_ref[...] + b2_ref[...]
    al = a_ref[0, 0]
    f2 = jnp.where(a2 >= 0, a2, al * a2)
    out_ref[...] = jnp.dot(f2, fw_ref[...], preferred_element_type=jnp.float32) + fb_ref[...]


def _tc3(p, t2, dinv, b2, fw, fb, a):
    return pl.pallas_call(
        _tc3_body,
        grid=(_NBLK,),
        in_specs=[
            pl.BlockSpec((NC, _BN, D), lambda i: (0, i, 0)),
            pl.BlockSpec((_BN, H2), lambda i: (i, 0)),
            pl.BlockSpec((_BN, 8), lambda i: (i, 0)),
            pl.BlockSpec((1, H2), lambda i: (0, 0)),
            pl.BlockSpec((H2, 128), lambda i: (0, 0)),
            pl.BlockSpec((1, 128), lambda i: (0, 0)),
            pl.BlockSpec((1, 1), lambda i: (0, 0)),
        ],
        out_specs=pl.BlockSpec((_BN, 128), lambda i: (i, 0)),
        out_shape=jax.ShapeDtypeStruct((N, 128), jnp.float32),
    )(p, t2, dinv, b2, fw, fb, a)


def kernel(x, edge_index, W1, b1, W2, b2, prelu_a, fc_W, fc_b):
    dst_r = edge_index[1].reshape(NW, EPW // KD, KD)
    src_c = edge_index[0].reshape(NW, NCH, CH, K)
    dst_c = edge_index[1].reshape(NW, NCH, CH, K)
    a = jnp.reshape(prelu_a, (1, 1)).astype(jnp.float32)
    b1r = b1.reshape(1, H1)
    b2r = b2.reshape(1, H2)
    fw_pad = jnp.zeros((H2, 128), jnp.float32).at[:, :OUT].set(fc_W)
    fb_pad = jnp.zeros((1, 128), jnp.float32).at[:, :OUT].set(fc_b)

    degp = _sc_degree(dst_r)
    dinv, u1 = _tc1(jnp.transpose(degp), x)
    p1 = _sc_scatter(u1, src_c, dst_c)
    t2, u2 = _tc2(p1, x, dinv, W1, b1r, W2, a)
    p2 = _sc_scatter(u2, src_c, dst_c)
    logits_pad = _tc3(p2, t2, dinv, b2r, fw_pad, fb_pad, a)
    return logits_pad[:, :OUT]
